# baseline (speedup 1.0000x reference)
import functools

import jax
import jax.numpy as jnp
from jax import lax
from jax.experimental import pallas as pl
from jax.experimental.pallas import tpu as pltpu
from jax.experimental.pallas import tpu_sc as plsc

N_NODES = 10000
NPAD = 10240
D = 128
N_EDGES = 320000

NC = 2
NS = 16
NW = NC * NS
EPW = N_EDGES // NW
CHUNK = 40
ITERS = EPW // CHUNK
ROWS_PER_TILE = NPAD // NS
ZROWS = 128

_mesh = plsc.VectorSubcoreMesh(core_axis_name="c", subcore_axis_name="s",
                               num_cores=NC, num_subcores=NS)


def _zero_fill(buf, nrows, ncols):
    zero = jnp.zeros((16,), jnp.float32)

    def body(i, carry):
        for j in range(ncols // 16):
            buf[i, pl.ds(j * 16, 16)] = zero
        return carry

    lax.fori_loop(0, nrows, body, 0)


DEG_RING = 5
DEG_DI = 2


@functools.partial(
    pl.kernel,
    out_type=jax.ShapeDtypeStruct((NC * NPAD, D), jnp.float32),
    mesh=_mesh,
    scratch_types=[
        [pltpu.VMEM((CHUNK,), jnp.int32) for _ in range(DEG_RING)],
        pltpu.VMEM((CHUNK, D), jnp.float32),
        pltpu.VMEM_SHARED((NPAD, D), jnp.float32),
        [pltpu.SemaphoreType.DMA for _ in range(DEG_RING)],
        [pltpu.SemaphoreType.DMA for _ in range(DEG_RING)],
    ],
)
def _sc_deg(dst_hbm, out, didxs, ones, acc, isems, ssems):
    cid = lax.axis_index("c")
    sid = lax.axis_index("s")
    wid = sid * NC + cid
    base = wid * EPW
    row0 = sid * ROWS_PER_TILE

    one = jnp.ones((16,), jnp.float32)

    _zero_fill(ones, CHUNK, D)
    for k in range(ROWS_PER_TILE // CHUNK):
        pltpu.async_copy(ones, acc.at[pl.ds(row0 + k * CHUNK, CHUNK)],
                         isems[0])
    for k in range(ROWS_PER_TILE // CHUNK):
        pltpu.make_async_copy(ones, acc.at[pl.ds(row0, CHUNK)],
                              isems[0]).wait()

    def refill(i, carry):
        for j in range(D // 16):
            ones[i, pl.ds(j * 16, 16)] = one
        return carry

    lax.fori_loop(0, CHUNK, refill, 0)
    plsc.subcore_barrier()

    def idx_start(j, b):
        pltpu.async_copy(dst_hbm.at[pl.ds(base + j * CHUNK, CHUNK)],
                         didxs[b], isems[b])

    def idx_wait(b):
        pltpu.make_async_copy(dst_hbm.at[pl.ds(base, CHUNK)],
                              didxs[b], isems[b]).wait()

    def scatter_wait(b):
        pltpu.make_async_copy(ones, acc.at[didxs[b]], ssems[b]).wait()

    for j in range(DEG_DI):
        idx_start(j, j)

    @pl.loop(0, ITERS, step=DEG_RING)
    def _(i0):
        for b in range(DEG_RING):
            i = i0 + b
            bi = (b + DEG_DI) % DEG_RING
            idx_wait(b)
            pltpu.async_copy(ones, acc.at[didxs[b]], ssems[b], add=True)

            @pl.when(i + DEG_DI < ITERS)
            def _():
                @pl.when(i + DEG_DI >= DEG_RING)
                def _():
                    scatter_wait(bi)

                idx_start(i + DEG_DI, bi)

    for b in range(DEG_RING):
        scatter_wait(b)

    plsc.subcore_barrier()
    pltpu.sync_copy(acc.at[pl.ds(row0, ROWS_PER_TILE)],
                    out.at[pl.ds(cid * NPAD + row0, ROWS_PER_TILE)])


RING = 5
DI = 2
DG = 1


@functools.partial(
    pl.kernel,
    out_type=jax.ShapeDtypeStruct((NC * NPAD, D), jnp.float32),
    mesh=_mesh,
    scratch_types=[
        [pltpu.VMEM((CHUNK,), jnp.int32) for _ in range(RING)],
        [pltpu.VMEM((CHUNK,), jnp.int32) for _ in range(RING)],
        [pltpu.VMEM((CHUNK, D), jnp.float32) for _ in range(RING)],
        pltpu.VMEM_SHARED((NPAD, D), jnp.float32),
        [pltpu.SemaphoreType.DMA for _ in range(RING)],
        [pltpu.SemaphoreType.DMA for _ in range(RING)],
        [pltpu.SemaphoreType.DMA for _ in range(RING)],
    ],
)
def _sc_agg(g_hbm, src_hbm, dst_hbm, out,
            sidxs, didxs, rowss, acc, isems, gsems, ssems):
    cid = lax.axis_index("c")
    sid = lax.axis_index("s")
    wid = sid * NC + cid
    base = wid * EPW
    row0 = sid * ROWS_PER_TILE

    _zero_fill(rowss[0], CHUNK, D)
    for k in range(ROWS_PER_TILE // CHUNK):
        pltpu.async_copy(rowss[0], acc.at[pl.ds(row0 + k * CHUNK, CHUNK)],
                         isems[0])
    for k in range(ROWS_PER_TILE // CHUNK):
        pltpu.make_async_copy(rowss[0], acc.at[pl.ds(row0, CHUNK)],
                              isems[0]).wait()
    plsc.subcore_barrier()

    def idx_start(j, b):
        pltpu.async_copy(src_hbm.at[pl.ds(base + j * CHUNK, CHUNK)],
                         sidxs[b], isems[b])
        pltpu.async_copy(dst_hbm.at[pl.ds(base + j * CHUNK, CHUNK)],
                         didxs[b], isems[b])

    def idx_wait(b):
        pltpu.make_async_copy(src_hbm.at[pl.ds(base, CHUNK)],
                              sidxs[b], isems[b]).wait()
        pltpu.make_async_copy(dst_hbm.at[pl.ds(base, CHUNK)],
                              didxs[b], isems[b]).wait()

    def gather_start(b):
        pltpu.async_copy(g_hbm.at[sidxs[b]], rowss[b], gsems[b])

    def gather_wait(b):
        pltpu.make_async_copy(g_hbm.at[sidxs[b]], rowss[b], gsems[b]).wait()

    def scatter_wait(b):
        pltpu.make_async_copy(rowss[b], acc.at[didxs[b]], ssems[b]).wait()

    for j in range(DI):
        idx_start(j, j)
    for j in range(DG):
        idx_wait(j)
        gather_start(j)

    @pl.loop(0, ITERS, step=RING)
    def _(i0):
        for b in range(RING):
            i = i0 + b
            bg = (b + DG) % RING
            bi = (b + DI) % RING
            gather_wait(b)
            pltpu.async_copy(rowss[b], acc.at[didxs[b]], ssems[b], add=True)

            @pl.when(i + DI < ITERS)
            def _():
                @pl.when(i + DI >= RING)
                def _():
                    scatter_wait(bi)

                idx_start(i + DI, bi)

            @pl.when(i + DG < ITERS)
            def _():
                idx_wait(bg)
                gather_start(bg)

    for b in range(RING):
        scatter_wait(b)

    plsc.subcore_barrier()
    pltpu.sync_copy(acc.at[pl.ds(row0, ROWS_PER_TILE)],
                    out.at[pl.ds(cid * NPAD + row0, ROWS_PER_TILE)])


MB = 1024


def _tc_mm_body(x_ref, w_ref, h_ref):
    h_ref[...] = jnp.dot(x_ref[...], w_ref[...],
                         preferred_element_type=jnp.float32)


def _tc_mm(x, w1):
    return pl.pallas_call(
        _tc_mm_body,
        grid=(NPAD // MB,),
        in_specs=[
            pl.BlockSpec((MB, D), lambda m: (m, 0)),
            pl.BlockSpec((D, D), lambda m: (0, 0)),
        ],
        out_specs=pl.BlockSpec((MB, D), lambda m: (m, 0)),
        out_shape=jax.ShapeDtypeStruct((NPAD, D), jnp.float32),
    )(x, w1)


def _tc_scale_body(d0_ref, d1_ref, h_ref, g_ref, dis_ref):
    deg = d0_ref[:, 0:1] + d1_ref[:, 0:1] + 1.0
    dis = lax.rsqrt(deg)
    dis_ref[...] = dis
    g_ref[...] = h_ref[...] * dis


def _tc_scale(d0, d1, h):
    return pl.pallas_call(
        _tc_scale_body,
        grid=(NPAD // MB,),
        in_specs=[
            pl.BlockSpec((MB, D), lambda m: (m, 0)),
            pl.BlockSpec((MB, D), lambda m: (m, 0)),
            pl.BlockSpec((MB, D), lambda m: (m, 0)),
        ],
        out_specs=[
            pl.BlockSpec((MB, D), lambda m: (m, 0)),
            pl.BlockSpec((MB, 1), lambda m: (m, 0)),
        ],
        out_shape=[
            jax.ShapeDtypeStruct((NPAD, D), jnp.float32),
            jax.ShapeDtypeStruct((NPAD, 1), jnp.float32),
        ],
    )(d0, d1, h)


def _tc_mid_body(a0_ref, a1_ref, g_ref, dis_ref, b_ref, w_ref, out_ref):
    dis = dis_ref[...]
    pre = dis * (a0_ref[...] + a1_ref[...] + g_ref[...]) + b_ref[...]
    r = jnp.maximum(pre, 0.0)
    out_ref[...] = jnp.dot(r, w_ref[...],
                           preferred_element_type=jnp.float32) * dis


def _tc_mid(a0, a1, g, dis, b1, w2):
    return pl.pallas_call(
        _tc_mid_body,
        grid=(NPAD // MB,),
        in_specs=[
            pl.BlockSpec((MB, D), lambda m: (m, 0)),
            pl.BlockSpec((MB, D), lambda m: (m, 0)),
            pl.BlockSpec((MB, D), lambda m: (m, 0)),
            pl.BlockSpec((MB, 1), lambda m: (m, 0)),
            pl.BlockSpec((1, D), lambda m: (0, 0)),
            pl.BlockSpec((D, D), lambda m: (0, 0)),
        ],
        out_specs=pl.BlockSpec((MB, D), lambda m: (m, 0)),
        out_shape=jax.ShapeDtypeStruct((NPAD, D), jnp.float32),
    )(a0, a1, g, dis, b1, w2)


def _tc_final_body(a0_ref, a1_ref, g_ref, dis_ref, b_ref, out_ref):
    pre = dis_ref[...] * (a0_ref[...] + a1_ref[...] + g_ref[...]) + b_ref[...]
    out_ref[...] = jnp.maximum(pre, 0.0)


def _tc_final(a0, a1, g, dis, b2):
    return pl.pallas_call(
        _tc_final_body,
        grid=(NPAD // MB,),
        in_specs=[
            pl.BlockSpec((MB, D), lambda m: (m, 0)),
            pl.BlockSpec((MB, D), lambda m: (m, 0)),
            pl.BlockSpec((MB, D), lambda m: (m, 0)),
            pl.BlockSpec((MB, 1), lambda m: (m, 0)),
            pl.BlockSpec((1, D), lambda m: (0, 0)),
        ],
        out_specs=pl.BlockSpec((MB, D), lambda m: (m, 0)),
        out_shape=jax.ShapeDtypeStruct((NPAD, D), jnp.float32),
    )(a0, a1, g, dis, b2)


def kernel(x, edge_index, batch, W1, b1, W2, b2):
    src = edge_index[0].astype(jnp.int32)
    dst = edge_index[1].astype(jnp.int32)
    xp = jnp.pad(x, ((0, NPAD - N_NODES), (0, 0)))

    h1 = _tc_mm(xp, W1)
    d = _sc_deg(dst)
    g1, dis = _tc_scale(d[:NPAD], d[NPAD:], h1)
    a = _sc_agg(g1, src, dst)
    g2 = _tc_mid(a[:NPAD], a[NPAD:], g1, dis, b1.reshape(1, D), W2)
    c = _sc_agg(g2, src, dst)
    return _tc_final(c[:NPAD], c[NPAD:], g2, dis, b2.reshape(1, D))[:N_NODES]


# speedup vs baseline: 1.4581x; 1.4581x over previous
import functools

import jax
import jax.numpy as jnp
from jax import lax
from jax.experimental import pallas as pl
from jax.experimental.pallas import tpu as pltpu
from jax.experimental.pallas import tpu_sc as plsc

N_NODES = 10000
NPAD = 10240
D = 128
N_EDGES = 320000

NC = 2
NS = 16
NW = NC * NS
EPW = N_EDGES // NW
CHUNK = 40
ITERS = EPW // CHUNK
ROWS_PER_TILE = NPAD // NS
ZROWS = 128

_mesh = plsc.VectorSubcoreMesh(core_axis_name="c", subcore_axis_name="s",
                               num_cores=NC, num_subcores=NS)


def _zero_fill(buf, nrows, ncols):
    zero = jnp.zeros((16,), jnp.float32)

    def body(i, carry):
        for j in range(ncols // 16):
            buf[i, pl.ds(j * 16, 16)] = zero
        return carry

    lax.fori_loop(0, nrows, body, 0)


DEG_RING = 5
DEG_DI = 3


@functools.partial(
    pl.kernel,
    out_type=jax.ShapeDtypeStruct((NC * NPAD, D), jnp.float32),
    mesh=_mesh,
    scratch_types=[
        [pltpu.VMEM((CHUNK,), jnp.int32) for _ in range(DEG_RING)],
        pltpu.VMEM((CHUNK, D), jnp.float32),
        pltpu.VMEM_SHARED((NPAD, D), jnp.float32),
        [pltpu.SemaphoreType.DMA for _ in range(DEG_RING)],
        [pltpu.SemaphoreType.DMA for _ in range(DEG_RING)],
    ],
)
def _sc_deg(dst_hbm, out, didxs, ones, acc, isems, ssems):
    cid = lax.axis_index("c")
    sid = lax.axis_index("s")
    wid = sid * NC + cid
    base = wid * EPW
    row0 = sid * ROWS_PER_TILE

    one = jnp.ones((16,), jnp.float32)

    _zero_fill(ones, CHUNK, D)
    for k in range(ROWS_PER_TILE // CHUNK):
        pltpu.async_copy(ones, acc.at[pl.ds(row0 + k * CHUNK, CHUNK)],
                         isems[0])
    for k in range(ROWS_PER_TILE // CHUNK):
        pltpu.make_async_copy(ones, acc.at[pl.ds(row0, CHUNK)],
                              isems[0]).wait()

    def refill(i, carry):
        for j in range(D // 16):
            ones[i, pl.ds(j * 16, 16)] = one
        return carry

    lax.fori_loop(0, CHUNK, refill, 0)
    plsc.subcore_barrier()

    def idx_start(j, b):
        pltpu.async_copy(dst_hbm.at[pl.ds(base + j * CHUNK, CHUNK)],
                         didxs[b], isems[b])

    def idx_wait(b):
        pltpu.make_async_copy(dst_hbm.at[pl.ds(base, CHUNK)],
                              didxs[b], isems[b]).wait()

    def scatter_wait(b):
        pltpu.make_async_copy(ones, acc.at[didxs[b]], ssems[b]).wait()

    for j in range(DEG_DI):
        idx_start(j, j)

    @pl.loop(0, ITERS, step=DEG_RING)
    def _(i0):
        for b in range(DEG_RING):
            i = i0 + b
            bi = (b + DEG_DI) % DEG_RING
            idx_wait(b)
            pltpu.async_copy(ones, acc.at[didxs[b]], ssems[b], add=True)

            @pl.when(i + DEG_DI < ITERS)
            def _():
                @pl.when(i + DEG_DI >= DEG_RING)
                def _():
                    scatter_wait(bi)

                idx_start(i + DEG_DI, bi)

    for b in range(DEG_RING):
        scatter_wait(b)

    plsc.subcore_barrier()
    pltpu.sync_copy(acc.at[pl.ds(row0, ROWS_PER_TILE)],
                    out.at[pl.ds(cid * NPAD + row0, ROWS_PER_TILE)])


RING = 5
DI = 3
DG = 2


@functools.partial(
    pl.kernel,
    out_type=jax.ShapeDtypeStruct((NC * NPAD, D), jnp.float32),
    mesh=_mesh,
    scratch_types=[
        [pltpu.VMEM((CHUNK,), jnp.int32) for _ in range(RING)],
        [pltpu.VMEM((CHUNK,), jnp.int32) for _ in range(RING)],
        [pltpu.VMEM((CHUNK, D), jnp.float32) for _ in range(RING)],
        pltpu.VMEM_SHARED((NPAD, D), jnp.float32),
        [pltpu.SemaphoreType.DMA for _ in range(RING)],
        [pltpu.SemaphoreType.DMA for _ in range(RING)],
        [pltpu.SemaphoreType.DMA for _ in range(RING)],
    ],
)
def _sc_agg(g_hbm, src_hbm, dst_hbm, out,
            sidxs, didxs, rowss, acc, isems, gsems, ssems):
    cid = lax.axis_index("c")
    sid = lax.axis_index("s")
    wid = sid * NC + cid
    base = wid * EPW
    row0 = sid * ROWS_PER_TILE

    _zero_fill(rowss[0], CHUNK, D)
    for k in range(ROWS_PER_TILE // CHUNK):
        pltpu.async_copy(rowss[0], acc.at[pl.ds(row0 + k * CHUNK, CHUNK)],
                         isems[0])
    for k in range(ROWS_PER_TILE // CHUNK):
        pltpu.make_async_copy(rowss[0], acc.at[pl.ds(row0, CHUNK)],
                              isems[0]).wait()
    plsc.subcore_barrier()

    def idx_start(j, b):
        pltpu.async_copy(src_hbm.at[pl.ds(base + j * CHUNK, CHUNK)],
                         sidxs[b], isems[b])
        pltpu.async_copy(dst_hbm.at[pl.ds(base + j * CHUNK, CHUNK)],
                         didxs[b], isems[b])

    def idx_wait(b):
        pltpu.make_async_copy(src_hbm.at[pl.ds(base, CHUNK)],
                              sidxs[b], isems[b]).wait()
        pltpu.make_async_copy(dst_hbm.at[pl.ds(base, CHUNK)],
                              didxs[b], isems[b]).wait()

    def gather_start(b):
        pltpu.async_copy(g_hbm.at[sidxs[b]], rowss[b], gsems[b])

    def gather_wait(b):
        pltpu.make_async_copy(g_hbm.at[sidxs[b]], rowss[b], gsems[b]).wait()

    def scatter_wait(b):
        pltpu.make_async_copy(rowss[b], acc.at[didxs[b]], ssems[b]).wait()

    for j in range(DI):
        idx_start(j, j)
    for j in range(DG):
        idx_wait(j)
        gather_start(j)

    @pl.loop(0, ITERS, step=RING)
    def _(i0):
        for b in range(RING):
            i = i0 + b
            bg = (b + DG) % RING
            bi = (b + DI) % RING
            gather_wait(b)
            pltpu.async_copy(rowss[b], acc.at[didxs[b]], ssems[b], add=True)

            @pl.when(i + DI < ITERS)
            def _():
                @pl.when(i + DI >= RING)
                def _():
                    scatter_wait(bi)

                idx_start(i + DI, bi)

            @pl.when(i + DG < ITERS)
            def _():
                idx_wait(bg)
                gather_start(bg)

    for b in range(RING):
        scatter_wait(b)

    plsc.subcore_barrier()
    pltpu.sync_copy(acc.at[pl.ds(row0, ROWS_PER_TILE)],
                    out.at[pl.ds(cid * NPAD + row0, ROWS_PER_TILE)])


MB = 1024


def _tc_mm_body(x_ref, w_ref, h_ref):
    h_ref[...] = jnp.dot(x_ref[...], w_ref[...],
                         preferred_element_type=jnp.float32)


def _tc_mm(x, w1):
    return pl.pallas_call(
        _tc_mm_body,
        grid=(NPAD // MB,),
        in_specs=[
            pl.BlockSpec((MB, D), lambda m: (m, 0)),
            pl.BlockSpec((D, D), lambda m: (0, 0)),
        ],
        out_specs=pl.BlockSpec((MB, D), lambda m: (m, 0)),
        out_shape=jax.ShapeDtypeStruct((NPAD, D), jnp.float32),
    )(x, w1)


def _tc_scale_body(d0_ref, d1_ref, h_ref, g_ref, dis_ref):
    deg = d0_ref[:, 0:1] + d1_ref[:, 0:1] + 1.0
    dis = lax.rsqrt(deg)
    dis_ref[...] = dis
    g_ref[...] = h_ref[...] * dis


def _tc_scale(d0, d1, h):
    return pl.pallas_call(
        _tc_scale_body,
        grid=(NPAD // MB,),
        in_specs=[
            pl.BlockSpec((MB, D), lambda m: (m, 0)),
            pl.BlockSpec((MB, D), lambda m: (m, 0)),
            pl.BlockSpec((MB, D), lambda m: (m, 0)),
        ],
        out_specs=[
            pl.BlockSpec((MB, D), lambda m: (m, 0)),
            pl.BlockSpec((MB, 1), lambda m: (m, 0)),
        ],
        out_shape=[
            jax.ShapeDtypeStruct((NPAD, D), jnp.float32),
            jax.ShapeDtypeStruct((NPAD, 1), jnp.float32),
        ],
    )(d0, d1, h)


def _tc_mid_body(a0_ref, a1_ref, g_ref, dis_ref, b_ref, w_ref, out_ref):
    dis = dis_ref[...]
    pre = dis * (a0_ref[...] + a1_ref[...] + g_ref[...]) + b_ref[...]
    r = jnp.maximum(pre, 0.0)
    out_ref[...] = jnp.dot(r, w_ref[...],
                           preferred_element_type=jnp.float32) * dis


def _tc_mid(a0, a1, g, dis, b1, w2):
    return pl.pallas_call(
        _tc_mid_body,
        grid=(NPAD // MB,),
        in_specs=[
            pl.BlockSpec((MB, D), lambda m: (m, 0)),
            pl.BlockSpec((MB, D), lambda m: (m, 0)),
            pl.BlockSpec((MB, D), lambda m: (m, 0)),
            pl.BlockSpec((MB, 1), lambda m: (m, 0)),
            pl.BlockSpec((1, D), lambda m: (0, 0)),
            pl.BlockSpec((D, D), lambda m: (0, 0)),
        ],
        out_specs=pl.BlockSpec((MB, D), lambda m: (m, 0)),
        out_shape=jax.ShapeDtypeStruct((NPAD, D), jnp.float32),
    )(a0, a1, g, dis, b1, w2)


def _tc_final_body(a0_ref, a1_ref, g_ref, dis_ref, b_ref, out_ref):
    pre = dis_ref[...] * (a0_ref[...] + a1_ref[...] + g_ref[...]) + b_ref[...]
    out_ref[...] = jnp.maximum(pre, 0.0)


def _tc_final(a0, a1, g, dis, b2):
    return pl.pallas_call(
        _tc_final_body,
        grid=(NPAD // MB,),
        in_specs=[
            pl.BlockSpec((MB, D), lambda m: (m, 0)),
            pl.BlockSpec((MB, D), lambda m: (m, 0)),
            pl.BlockSpec((MB, D), lambda m: (m, 0)),
            pl.BlockSpec((MB, 1), lambda m: (m, 0)),
            pl.BlockSpec((1, D), lambda m: (0, 0)),
        ],
        out_specs=pl.BlockSpec((MB, D), lambda m: (m, 0)),
        out_shape=jax.ShapeDtypeStruct((NPAD, D), jnp.float32),
    )(a0, a1, g, dis, b2)


def kernel(x, edge_index, batch, W1, b1, W2, b2):
    src = edge_index[0].astype(jnp.int32)
    dst = edge_index[1].astype(jnp.int32)
    xp = jnp.pad(x, ((0, NPAD - N_NODES), (0, 0)))

    h1 = _tc_mm(xp, W1)
    d = _sc_deg(dst)
    g1, dis = _tc_scale(d[:NPAD], d[NPAD:], h1)
    a = _sc_agg(g1, src, dst)
    g2 = _tc_mid(a[:NPAD], a[NPAD:], g1, dis, b1.reshape(1, D), W2)
    c = _sc_agg(g2, src, dst)
    return _tc_final(c[:NPAD], c[NPAD:], g2, dis, b2.reshape(1, D))[:N_NODES]
